# baseline (speedup 1.0000x reference)
import functools
import math

import jax
import jax.numpy as jnp
from jax import lax
from jax.experimental import pallas as pl
from jax.experimental.pallas import tpu as pltpu
from jax.experimental.pallas import tpu_sc as plsc

EMBED, HID, PAT = 384, 256, 512
MEM, B, K = 131072, 16384, 32
NC, NS, L = 2, 16, 16
NW = NC * NS
QW = B // NW
SPT = MEM // NS
INV_SQRT_PAT = 1.0 / math.sqrt(PAT)
ENC_BLK = 512


def _enc_body(emb, w1, b1, w2, b2, g, bta, out):
    h = jnp.dot(emb[...], w1[...], precision=lax.Precision.HIGHEST,
                preferred_element_type=jnp.float32) + b1[...]
    h = jnp.maximum(h, 0.0)
    p = jnp.dot(h, w2[...], precision=lax.Precision.HIGHEST,
                preferred_element_type=jnp.float32) + b2[...]
    mu = jnp.mean(p, axis=-1, keepdims=True)
    var = jnp.mean((p - mu) ** 2, axis=-1, keepdims=True)
    out[...] = (p - mu) * lax.rsqrt(var + 1e-5) * g[...] + bta[...]


def _encode(emb, w1, b1, w2, b2, g, bta):
    return pl.pallas_call(
        _enc_body,
        grid=(B // ENC_BLK,),
        in_specs=[
            pl.BlockSpec((ENC_BLK, EMBED), lambda i: (i, 0)),
            pl.BlockSpec((EMBED, HID), lambda i: (0, 0)),
            pl.BlockSpec((1, HID), lambda i: (0, 0)),
            pl.BlockSpec((HID, PAT), lambda i: (0, 0)),
            pl.BlockSpec((1, PAT), lambda i: (0, 0)),
            pl.BlockSpec((1, PAT), lambda i: (0, 0)),
            pl.BlockSpec((1, PAT), lambda i: (0, 0)),
        ],
        out_specs=pl.BlockSpec((ENC_BLK, PAT), lambda i: (i, 0)),
        out_shape=jax.ShapeDtypeStruct((B, PAT), jnp.float32),
    )(emb, w1, b1, w2, b2, g, bta)


def _sc_body(mem_hbm, pat_hbm, widx_hbm, ridx_hbm, out_hbm, owner_sh,
             ridx_v, widx_v, own_v, sortbuf,
             bidx, comb, owb, fixidx, roweff, pbv, accv,
             sem0, sem1, sem2, sem3):
    cid = lax.axis_index("c")
    sid = lax.axis_index("s")
    wid = sid * NC + cid
    qbase = wid * QW
    iota = lax.iota(jnp.int32, L)
    negv = jnp.full((L,), -1, jnp.int32)
    zf = jnp.zeros((L,), jnp.float32)

    sbase = sid * SPT

    def _init(ii, _):
        own_v[pl.ds(ii * L, L)] = negv
        return 0

    lax.fori_loop(0, SPT // L, _init, 0)
    sortbuf[pl.ds(L, L)] = negv
    for oc in range(B // 4096):
        pltpu.sync_copy(widx_hbm.at[pl.ds(oc * 4096, 4096)], widx_v)

        def _scan(g, _, oc=oc):
            idx = widx_v[pl.ds(g * L, L)]
            bv = oc * 4096 + g * L + iota
            key = idx * 16384 + bv
            sk, _sv = plsc.sort_key_val(key, key)
            slot = lax.shift_right_logical(sk, 14)
            bval = lax.bitwise_and(sk, 16383)
            sortbuf[pl.ds(0, L)] = slot
            nxt = sortbuf[pl.ds(1, L)]
            keep = slot != nxt
            inr = (slot >= sbase) & (slot < sbase + SPT)
            plsc.store_scatter(own_v, [slot - sbase], bval, mask=keep & inr)
            return 0

        lax.fori_loop(0, 4096 // L, _scan, 0)
    pltpu.sync_copy(own_v, owner_sh.at[pl.ds(sbase, SPT)])
    plsc.subcore_barrier()

    pltpu.sync_copy(ridx_hbm.at[pl.ds(qbase * K, QW * K)], ridx_v)

    ones_i = jnp.full((L,), 1, jnp.int32)

    def _do_chunk(c, _):
        for v in range(K // L):
            bidx[pl.ds(v * L, L)] = ridx_v[pl.ds(c * K + v * L, L)]
        pltpu.async_copy(mem_hbm.at[bidx], comb.at[pl.ds(0, K)], sem0)
        pltpu.async_copy(pat_hbm.at[qbase + c], pbv, sem1)
        pltpu.async_copy(owner_sh.at[bidx], owb, sem2)
        pltpu.make_async_copy(owner_sh.at[bidx], owb, sem2).wait()

        fixidx[pl.ds(0, L)] = qbase + iota
        tot = 0
        rvs = []
        for v in range(K // L):
            owv = owb[pl.ds(v * L, L)]
            mval = owv >= 0
            cnt = plsc.cumsum(ones_i, mask=mval)
            pos = tot + cnt - 1
            valid = mval & (pos < L)
            plsc.store_scatter(fixidx, [pos], owv, mask=valid)
            rv = jnp.where(valid, K + pos, v * L + iota)
            roweff[pl.ds(v * L, L)] = rv
            rvs.append(rv)
            tot = tot + jnp.sum(jnp.where(mval, 1, 0))
        r0, r1 = rvs
        pltpu.async_copy(pat_hbm.at[fixidx], comb.at[pl.ds(K, L)], sem3)
        pltpu.make_async_copy(mem_hbm.at[bidx], comb.at[pl.ds(0, K)],
                              sem0).wait()
        pltpu.make_async_copy(pat_hbm.at[qbase + c], pbv, sem1).wait()
        pltpu.make_async_copy(pat_hbm.at[fixidx], comb.at[pl.ds(K, L)],
                              sem3).wait()

        def _sc_loop(wc, sacc):
            a0, a1 = sacc
            pch = pbv[pl.ds(wc * L, L)]
            for jj in range(L):
                w = wc * L + jj
                pv = pch[jj]
                word = jnp.full((L,), w, jnp.int32)
                v0 = plsc.load_gather(comb, [r0, word])
                v1 = plsc.load_gather(comb, [r1, word])
                a0 = a0 + v0 * pv
                a1 = a1 + v1 * pv
            return a0, a1

        s0, s1 = lax.fori_loop(0, PAT // L, _sc_loop, (zf, zf))
        s0 = s0 * INV_SQRT_PAT
        s1 = s1 * INV_SQRT_PAT
        m = jnp.maximum(jnp.max(s0), jnp.max(s1))
        e0 = jnp.exp(s0 - m)
        e1 = jnp.exp(s1 - m)
        lv = jnp.full((L,), jnp.sum(e0) + jnp.sum(e1), jnp.float32)
        w0 = e0 / lv
        w1 = e1 / lv

        def _zero(ii, _):
            accv[pl.ds(ii * L, L)] = zf
            return 0

        lax.fori_loop(0, PAT // L, _zero, 0)

        def _p2(k, wcarry):
            wv0, wv1, rv0, rv1 = wcarry
            lane = k % L
            wk = jnp.sum(jnp.where(
                iota == lane, jnp.where(k < L, wv0, wv1), 0.0))
            row = jnp.sum(jnp.where(
                iota == lane, jnp.where(k < L, rv0, rv1), 0))
            for cc in range(PAT // L):
                plsc.addupdate(accv.at[pl.ds(cc * L, L)],
                               comb[row, pl.ds(cc * L, L)] * wk)
            return wv0, wv1, rv0, rv1

        lax.fori_loop(0, K, _p2, (w0, w1, r0, r1))
        pltpu.sync_copy(accv, out_hbm.at[qbase + c])
        return 0

    lax.fori_loop(0, QW, _do_chunk, 0)


def _retrieve(memory_patterns, patterns, write_idx, read_idx_flat):
    mesh = plsc.VectorSubcoreMesh(
        core_axis_name="c", subcore_axis_name="s",
        num_cores=NC, num_subcores=NS)
    f = pl.kernel(
        _sc_body,
        out_type=jax.ShapeDtypeStruct((B, PAT), jnp.float32),
        mesh=mesh,
        compiler_params=pltpu.CompilerParams(needs_layout_passes=False),
        scratch_types=[
            pltpu.VMEM_SHARED((MEM,), jnp.int32),
            pltpu.VMEM((QW * K,), jnp.int32),
            pltpu.VMEM((4096,), jnp.int32),
            pltpu.VMEM((SPT,), jnp.int32),
            pltpu.VMEM((2 * L,), jnp.int32),
            pltpu.VMEM((K,), jnp.int32),
            pltpu.VMEM((K + L, PAT), jnp.float32),
            pltpu.VMEM((K,), jnp.int32),
            pltpu.VMEM((L,), jnp.int32),
            pltpu.VMEM((K,), jnp.int32),
            pltpu.VMEM((PAT,), jnp.float32),
            pltpu.VMEM((PAT,), jnp.float32),
            pltpu.SemaphoreType.DMA,
            pltpu.SemaphoreType.DMA,
            pltpu.SemaphoreType.DMA,
            pltpu.SemaphoreType.DMA,
        ],
    )
    return f(memory_patterns, patterns, write_idx, read_idx_flat)


def kernel(embeddings, memory_patterns, W_in, b_in, W_pat, b_pat,
           ln_gamma, ln_beta, write_idx, read_idx):
    patterns = _encode(
        embeddings, W_in, b_in.reshape(1, HID), W_pat, b_pat.reshape(1, PAT),
        ln_gamma.reshape(1, PAT), ln_beta.reshape(1, PAT))
    return _retrieve(
        memory_patterns, patterns,
        write_idx.astype(jnp.int32), read_idx.reshape(B * K).astype(jnp.int32))


# speedup vs baseline: 1.1021x; 1.1021x over previous
import functools
import math

import jax
import jax.numpy as jnp
from jax import lax
from jax.experimental import pallas as pl
from jax.experimental.pallas import tpu as pltpu
from jax.experimental.pallas import tpu_sc as plsc

EMBED, HID, PAT = 384, 256, 512
MEM, B, K = 131072, 16384, 32
NC, NS, L = 2, 16, 16
NW = NC * NS
QW = B // NW
SPT = MEM // NS
INV_SQRT_PAT = 1.0 / math.sqrt(PAT)
ENC_BLK = 512


def _enc_body(emb, w1, b1, w2, b2, g, bta, out):
    h = jnp.dot(emb[...], w1[...], precision=lax.Precision.HIGHEST,
                preferred_element_type=jnp.float32) + b1[...]
    h = jnp.maximum(h, 0.0)
    p = jnp.dot(h, w2[...], precision=lax.Precision.HIGHEST,
                preferred_element_type=jnp.float32) + b2[...]
    mu = jnp.mean(p, axis=-1, keepdims=True)
    var = jnp.mean((p - mu) ** 2, axis=-1, keepdims=True)
    out[...] = (p - mu) * lax.rsqrt(var + 1e-5) * g[...] + bta[...]


def _encode(emb, w1, b1, w2, b2, g, bta):
    return pl.pallas_call(
        _enc_body,
        grid=(B // ENC_BLK,),
        in_specs=[
            pl.BlockSpec((ENC_BLK, EMBED), lambda i: (i, 0)),
            pl.BlockSpec((EMBED, HID), lambda i: (0, 0)),
            pl.BlockSpec((1, HID), lambda i: (0, 0)),
            pl.BlockSpec((HID, PAT), lambda i: (0, 0)),
            pl.BlockSpec((1, PAT), lambda i: (0, 0)),
            pl.BlockSpec((1, PAT), lambda i: (0, 0)),
            pl.BlockSpec((1, PAT), lambda i: (0, 0)),
        ],
        out_specs=pl.BlockSpec((ENC_BLK, PAT), lambda i: (i, 0)),
        out_shape=jax.ShapeDtypeStruct((B, PAT), jnp.float32),
    )(emb, w1, b1, w2, b2, g, bta)


def _sc_body(mem_hbm, pat_hbm, widx_hbm, ridx_hbm, out_hbm, owner_sh,
             ridx_v, widx_v, own_v, sortbuf,
             bidx0, bidx1, bidx2,
             comb0, comb1, comb2,
             owb0, owb1, owb2,
             fxi0, fxi1, fxi2,
             rwe0, rwe1, rwe2,
             pbv0, pbv1, pbv2,
             acc0, acc1, acc2,
             smb0, smb1, smb2,
             smw0, smw1, smw2,
             smp0, smp1, smp2,
             smf0, smf1, smf2,
             smo0, smo1, smo2):
    bidxs = (bidx0, bidx1, bidx2)
    combs = (comb0, comb1, comb2)
    owbs = (owb0, owb1, owb2)
    fxis = (fxi0, fxi1, fxi2)
    rwes = (rwe0, rwe1, rwe2)
    pbvs = (pbv0, pbv1, pbv2)
    accs = (acc0, acc1, acc2)
    smbs = (smb0, smb1, smb2)
    smws = (smw0, smw1, smw2)
    smps = (smp0, smp1, smp2)
    smfs = (smf0, smf1, smf2)
    smos = (smo0, smo1, smo2)
    cid = lax.axis_index("c")
    sid = lax.axis_index("s")
    wid = sid * NC + cid
    qbase = wid * QW
    iota = lax.iota(jnp.int32, L)
    negv = jnp.full((L,), -1, jnp.int32)
    zf = jnp.zeros((L,), jnp.float32)

    sbase = sid * SPT

    def _init(ii, _):
        own_v[pl.ds(ii * L, L)] = negv
        return 0

    lax.fori_loop(0, SPT // L, _init, 0)
    sortbuf[pl.ds(L, L)] = negv
    for oc in range(B // 4096):
        pltpu.sync_copy(widx_hbm.at[pl.ds(oc * 4096, 4096)], widx_v)

        def _scan(g, _, oc=oc):
            idx = widx_v[pl.ds(g * L, L)]
            bv = oc * 4096 + g * L + iota
            key = idx * 16384 + bv
            sk, _sv = plsc.sort_key_val(key, key)
            slot = lax.shift_right_logical(sk, 14)
            bval = lax.bitwise_and(sk, 16383)
            sortbuf[pl.ds(0, L)] = slot
            nxt = sortbuf[pl.ds(1, L)]
            keep = slot != nxt
            inr = (slot >= sbase) & (slot < sbase + SPT)
            plsc.store_scatter(own_v, [slot - sbase], bval, mask=keep & inr)
            return 0

        lax.fori_loop(0, 4096 // L, _scan, 0)
    pltpu.sync_copy(own_v, owner_sh.at[pl.ds(sbase, SPT)])
    plsc.subcore_barrier()

    pltpu.sync_copy(ridx_hbm.at[pl.ds(qbase * K, QW * K)], ridx_v)

    ones_i = jnp.full((L,), 1, jnp.int32)

    def _issue_small(c, j):
        for v in range(K // L):
            bidxs[j][pl.ds(v * L, L)] = ridx_v[pl.ds(c * K + v * L, L)]
        pltpu.async_copy(owner_sh.at[bidxs[j]], owbs[j], smws[j])
        pltpu.async_copy(pat_hbm.at[qbase + c], pbvs[j], smps[j])
        pltpu.async_copy(mem_hbm.at[bidxs[j]], combs[j].at[pl.ds(0, K)],
                         smbs[j])

    def _wait_ow(j):
        pltpu.make_async_copy(owner_sh.at[bidxs[j]], owbs[j], smws[j]).wait()

    def _wait_pbv(c, j):
        pltpu.make_async_copy(pat_hbm.at[qbase + c], pbvs[j], smps[j]).wait()

    def _wait_bulk(j):
        pltpu.make_async_copy(mem_hbm.at[bidxs[j]],
                              combs[j].at[pl.ds(0, K)], smbs[j]).wait()

    def _issue_fix(j):
        fxis[j][pl.ds(0, L)] = qbase + iota
        tot = 0
        for v in range(K // L):
            owv = owbs[j][pl.ds(v * L, L)]
            mval = owv >= 0
            cnt = plsc.cumsum(ones_i, mask=mval)
            pos = tot + cnt - 1
            valid = mval & (pos < L)
            plsc.store_scatter(fxis[j], [pos], owv, mask=valid)
            rwes[j][pl.ds(v * L, L)] = jnp.where(valid, K + pos,
                                                 v * L + iota)
            tot = tot + jnp.sum(jnp.where(mval, 1, 0))
        pltpu.async_copy(pat_hbm.at[fxis[j]], combs[j].at[pl.ds(K, L)],
                         smfs[j])

    def _wait_fix(j):
        pltpu.make_async_copy(pat_hbm.at[fxis[j]],
                              combs[j].at[pl.ds(K, L)], smfs[j]).wait()

    for c0 in range(3):
        _issue_small(c0, c0)
    _wait_bulk(0)
    _wait_ow(0)
    _issue_fix(0)

    def _do_chunk(c, j):
        accv = accs[j]

        @pl.when(c + 1 < QW)
        def _():
            jn = (j + 1) % 3
            _wait_bulk(jn)
            _wait_ow(jn)
            _issue_fix(jn)

        _wait_fix(j)
        _wait_pbv(c, j)
        r0 = rwes[j][pl.ds(0, L)]
        r1 = rwes[j][pl.ds(L, L)]

        def _sc_loop(wc, sacc):
            a0, a1 = sacc
            pch = pbvs[j][pl.ds(wc * L, L)]
            for jj in range(L):
                w = wc * L + jj
                pv = pch[jj]
                word = jnp.full((L,), w, jnp.int32)
                v0 = plsc.load_gather(combs[j], [r0, word])
                v1 = plsc.load_gather(combs[j], [r1, word])
                a0 = a0 + v0 * pv
                a1 = a1 + v1 * pv
            return a0, a1

        s0, s1 = lax.fori_loop(0, PAT // L, _sc_loop, (zf, zf))
        s0 = s0 * INV_SQRT_PAT
        s1 = s1 * INV_SQRT_PAT
        m = jnp.maximum(jnp.max(s0), jnp.max(s1))
        e0 = jnp.exp(s0 - m)
        e1 = jnp.exp(s1 - m)
        lv = jnp.full((L,), jnp.sum(e0) + jnp.sum(e1), jnp.float32)
        w0 = e0 / lv
        w1 = e1 / lv

        @pl.when(c >= 3)
        def _():
            pltpu.make_async_copy(
                accv, out_hbm.at[qbase + c - 3], smos[j]).wait()

        def _zero(ii, _):
            accv[pl.ds(ii * L, L)] = zf
            return 0

        lax.fori_loop(0, PAT // L, _zero, 0)

        def _p2(k, wcarry):
            wv0, wv1, rv0, rv1 = wcarry
            lane = k % L
            wk = jnp.sum(jnp.where(
                iota == lane, jnp.where(k < L, wv0, wv1), 0.0))
            row = jnp.sum(jnp.where(
                iota == lane, jnp.where(k < L, rv0, rv1), 0))
            for cc in range(PAT // L):
                plsc.addupdate(accv.at[pl.ds(cc * L, L)],
                               combs[j][row, pl.ds(cc * L, L)] * wk)
            return wv0, wv1, rv0, rv1

        lax.fori_loop(0, K, _p2, (w0, w1, r0, r1))
        pltpu.async_copy(accv, out_hbm.at[qbase + c], smos[j])

        @pl.when(c + 3 < QW)
        def _():
            _issue_small(c + 3, j)

    def _chunk_body(i, _):
        for j in range(3):
            _do_chunk(i * 3 + j, j)
        return 0

    lax.fori_loop(0, (QW - 2) // 3, _chunk_body, 0)
    _do_chunk(jnp.int32(QW - 2), 0)
    _do_chunk(jnp.int32(QW - 1), 1)
    for j in range(3):
        pltpu.make_async_copy(
            accs[j], out_hbm.at[qbase + QW - 3 + j], smos[j]).wait()


def _retrieve(memory_patterns, patterns, write_idx, read_idx_flat):
    mesh = plsc.VectorSubcoreMesh(
        core_axis_name="c", subcore_axis_name="s",
        num_cores=NC, num_subcores=NS)
    f = pl.kernel(
        _sc_body,
        out_type=jax.ShapeDtypeStruct((B, PAT), jnp.float32),
        mesh=mesh,
        compiler_params=pltpu.CompilerParams(needs_layout_passes=False),
        scratch_types=[
            pltpu.VMEM_SHARED((MEM,), jnp.int32),
            pltpu.VMEM((QW * K,), jnp.int32),
            pltpu.VMEM((4096,), jnp.int32),
            pltpu.VMEM((SPT,), jnp.int32),
            pltpu.VMEM((2 * L,), jnp.int32),
        ] + [pltpu.VMEM((K,), jnp.int32)] * 3
          + [pltpu.VMEM((K + L, PAT), jnp.float32)] * 3
          + [pltpu.VMEM((K,), jnp.int32)] * 3
          + [pltpu.VMEM((L,), jnp.int32)] * 3
          + [pltpu.VMEM((K,), jnp.int32)] * 3
          + [pltpu.VMEM((PAT,), jnp.float32)] * 3
          + [pltpu.VMEM((PAT,), jnp.float32)] * 3
          + [pltpu.SemaphoreType.DMA] * 15,
    )
    return f(memory_patterns, patterns, write_idx, read_idx_flat)


def kernel(embeddings, memory_patterns, W_in, b_in, W_pat, b_pat,
           ln_gamma, ln_beta, write_idx, read_idx):
    patterns = _encode(
        embeddings, W_in, b_in.reshape(1, HID), W_pat, b_pat.reshape(1, PAT),
        ln_gamma.reshape(1, PAT), ln_beta.reshape(1, PAT))
    return _retrieve(
        memory_patterns, patterns,
        write_idx.astype(jnp.int32), read_idx.reshape(B * K).astype(jnp.int32))


# speedup vs baseline: 2.2255x; 2.0195x over previous
import functools
import math

import jax
import jax.numpy as jnp
from jax import lax
from jax.experimental import pallas as pl
from jax.experimental.pallas import tpu as pltpu
from jax.experimental.pallas import tpu_sc as plsc

EMBED, HID, PAT = 384, 256, 512
MEM, B, K = 131072, 16384, 32
NC, NS, L = 2, 16, 16
NW = NC * NS
QW = B // NW
SPT = MEM // NS
INV_SQRT_PAT = 1.0 / math.sqrt(PAT)
ENC_BLK = 512


def _enc_body(emb, w1, b1, w2, b2, g, bta, out):
    h = jnp.dot(emb[...], w1[...], precision=lax.Precision.HIGHEST,
                preferred_element_type=jnp.float32) + b1[...]
    h = jnp.maximum(h, 0.0)
    p = jnp.dot(h, w2[...], precision=lax.Precision.HIGHEST,
                preferred_element_type=jnp.float32) + b2[...]
    mu = jnp.mean(p, axis=-1, keepdims=True)
    var = jnp.mean((p - mu) ** 2, axis=-1, keepdims=True)
    out[...] = (p - mu) * lax.rsqrt(var + 1e-5) * g[...] + bta[...]


def _encode(emb, w1, b1, w2, b2, g, bta):
    return pl.pallas_call(
        _enc_body,
        grid=(B // ENC_BLK,),
        in_specs=[
            pl.BlockSpec((ENC_BLK, EMBED), lambda i: (i, 0)),
            pl.BlockSpec((EMBED, HID), lambda i: (0, 0)),
            pl.BlockSpec((1, HID), lambda i: (0, 0)),
            pl.BlockSpec((HID, PAT), lambda i: (0, 0)),
            pl.BlockSpec((1, PAT), lambda i: (0, 0)),
            pl.BlockSpec((1, PAT), lambda i: (0, 0)),
            pl.BlockSpec((1, PAT), lambda i: (0, 0)),
        ],
        out_specs=pl.BlockSpec((ENC_BLK, PAT), lambda i: (i, 0)),
        out_shape=jax.ShapeDtypeStruct((B, PAT), jnp.float32),
    )(emb, w1, b1, w2, b2, g, bta)


def _sc_body(mem_hbm, pat_hbm, widx_hbm, ridx_hbm, out_hbm, owner_sh,
             ridx_v, widx_v, own_v, sortbuf, scob,
             bidx0, bidx1, bidx2,
             comb0, comb1, comb2,
             owb0, owb1, owb2,
             fxi0, fxi1, fxi2,
             rwe0, rwe1, rwe2,
             pbv0, pbv1, pbv2,
             acc0, acc1, acc2,
             smb0, smb1, smb2,
             smw0, smw1, smw2,
             smp0, smp1, smp2,
             smf0, smf1, smf2,
             smo0, smo1, smo2):
    bidxs = (bidx0, bidx1, bidx2)
    combs = (comb0, comb1, comb2)
    owbs = (owb0, owb1, owb2)
    fxis = (fxi0, fxi1, fxi2)
    rwes = (rwe0, rwe1, rwe2)
    pbvs = (pbv0, pbv1, pbv2)
    accs = (acc0, acc1, acc2)
    smbs = (smb0, smb1, smb2)
    smws = (smw0, smw1, smw2)
    smps = (smp0, smp1, smp2)
    smfs = (smf0, smf1, smf2)
    smos = (smo0, smo1, smo2)
    cid = lax.axis_index("c")
    sid = lax.axis_index("s")
    wid = sid * NC + cid
    qbase = wid * QW
    iota = lax.iota(jnp.int32, L)
    negv = jnp.full((L,), -1, jnp.int32)
    zf = jnp.zeros((L,), jnp.float32)

    sbase = sid * SPT

    def _init(ii, _):
        own_v[pl.ds(ii * L, L)] = negv
        return 0

    lax.fori_loop(0, SPT // L, _init, 0)
    sortbuf[pl.ds(L, L)] = negv
    for oc in range(B // 4096):
        pltpu.sync_copy(widx_hbm.at[pl.ds(oc * 4096, 4096)], widx_v)

        def _scan(g, _, oc=oc):
            idx = widx_v[pl.ds(g * L, L)]
            bv = oc * 4096 + g * L + iota
            key = idx * 16384 + bv
            sk, _sv = plsc.sort_key_val(key, key)
            slot = lax.shift_right_logical(sk, 14)
            bval = lax.bitwise_and(sk, 16383)
            sortbuf[pl.ds(0, L)] = slot
            nxt = sortbuf[pl.ds(1, L)]
            keep = slot != nxt
            inr = (slot >= sbase) & (slot < sbase + SPT)
            plsc.store_scatter(own_v, [slot - sbase], bval, mask=keep & inr)
            return 0

        lax.fori_loop(0, 4096 // L, _scan, 0)
    pltpu.sync_copy(own_v, owner_sh.at[pl.ds(sbase, SPT)])
    plsc.subcore_barrier()

    pltpu.sync_copy(ridx_hbm.at[pl.ds(qbase * K, QW * K)], ridx_v)

    ones_i = jnp.full((L,), 1, jnp.int32)

    def _issue_small(c, j):
        for v in range(K // L):
            bidxs[j][pl.ds(v * L, L)] = ridx_v[pl.ds(c * K + v * L, L)]
        pltpu.async_copy(owner_sh.at[bidxs[j]], owbs[j], smws[j])
        pltpu.async_copy(pat_hbm.at[qbase + c], pbvs[j], smps[j])
        pltpu.async_copy(mem_hbm.at[bidxs[j]], combs[j].at[pl.ds(0, K)],
                         smbs[j])

    def _wait_ow(j):
        pltpu.make_async_copy(owner_sh.at[bidxs[j]], owbs[j], smws[j]).wait()

    def _wait_pbv(c, j):
        pltpu.make_async_copy(pat_hbm.at[qbase + c], pbvs[j], smps[j]).wait()

    def _wait_bulk(j):
        pltpu.make_async_copy(mem_hbm.at[bidxs[j]],
                              combs[j].at[pl.ds(0, K)], smbs[j]).wait()

    def _issue_fix(j):
        fxis[j][pl.ds(0, L)] = qbase + iota
        tot = 0
        for v in range(K // L):
            owv = owbs[j][pl.ds(v * L, L)]
            mval = owv >= 0
            cnt = plsc.cumsum(ones_i, mask=mval)
            pos = tot + cnt - 1
            valid = mval & (pos < L)
            plsc.store_scatter(fxis[j], [pos], owv, mask=valid)
            rwes[j][pl.ds(v * L, L)] = jnp.where(valid, K + pos,
                                                 v * L + iota)
            tot = tot + jnp.sum(jnp.where(mval, 1, 0))
        pltpu.async_copy(pat_hbm.at[fxis[j]], combs[j].at[pl.ds(K, L)],
                         smfs[j])

    def _wait_fix(j):
        pltpu.make_async_copy(pat_hbm.at[fxis[j]],
                              combs[j].at[pl.ds(K, L)], smfs[j]).wait()

    for c0 in range(3):
        _issue_small(c0, c0)
    _wait_bulk(0)
    _wait_ow(0)
    _issue_fix(0)

    def _do_chunk(c, j):
        accv = accs[j]

        @pl.when(c + 1 < QW)
        def _():
            jn = (j + 1) % 3
            _wait_bulk(jn)
            _wait_ow(jn)
            _issue_fix(jn)

        _wait_fix(j)
        _wait_pbv(c, j)
        r0 = rwes[j][pl.ds(0, L)]
        r1 = rwes[j][pl.ds(L, L)]

        def _sc_loop(k, wcarry):
            rv0, rv1 = wcarry
            lane = k % L
            row = jnp.sum(jnp.where(
                iota == lane, jnp.where(k < L, rv0, rv1), 0))
            a = zf
            for cc in range(PAT // L):
                a = a + (combs[j][row, pl.ds(cc * L, L)]
                         * pbvs[j][pl.ds(cc * L, L)])
            sc = jnp.sum(a)
            plsc.store_scatter(scob, [jnp.full((L,), k, jnp.int32)],
                               jnp.full((L,), sc, jnp.float32),
                               mask=iota == 0)
            return rv0, rv1

        lax.fori_loop(0, K, _sc_loop, (r0, r1))
        s0 = scob[pl.ds(0, L)] * INV_SQRT_PAT
        s1 = scob[pl.ds(L, L)] * INV_SQRT_PAT
        m = jnp.maximum(jnp.max(s0), jnp.max(s1))
        e0 = jnp.exp(s0 - m)
        e1 = jnp.exp(s1 - m)
        lv = jnp.full((L,), jnp.sum(e0) + jnp.sum(e1), jnp.float32)
        w0 = e0 / lv
        w1 = e1 / lv

        @pl.when(c >= 3)
        def _():
            pltpu.make_async_copy(
                accv, out_hbm.at[qbase + c - 3], smos[j]).wait()

        def _zero(ii, _):
            accv[pl.ds(ii * L, L)] = zf
            return 0

        lax.fori_loop(0, PAT // L, _zero, 0)

        def _p2(k, wcarry):
            wv0, wv1, rv0, rv1 = wcarry
            lane = k % L
            wk = jnp.sum(jnp.where(
                iota == lane, jnp.where(k < L, wv0, wv1), 0.0))
            row = jnp.sum(jnp.where(
                iota == lane, jnp.where(k < L, rv0, rv1), 0))
            for cc in range(PAT // L):
                plsc.addupdate(accv.at[pl.ds(cc * L, L)],
                               combs[j][row, pl.ds(cc * L, L)] * wk)
            return wv0, wv1, rv0, rv1

        lax.fori_loop(0, K, _p2, (w0, w1, r0, r1))
        pltpu.async_copy(accv, out_hbm.at[qbase + c], smos[j])

        @pl.when(c + 3 < QW)
        def _():
            _issue_small(c + 3, j)

    def _chunk_body(i, _):
        for j in range(3):
            _do_chunk(i * 3 + j, j)
        return 0

    lax.fori_loop(0, (QW - 2) // 3, _chunk_body, 0)
    _do_chunk(jnp.int32(QW - 2), 0)
    _do_chunk(jnp.int32(QW - 1), 1)
    for j in range(3):
        pltpu.make_async_copy(
            accs[j], out_hbm.at[qbase + QW - 3 + j], smos[j]).wait()


def _retrieve(memory_patterns, patterns, write_idx, read_idx_flat):
    mesh = plsc.VectorSubcoreMesh(
        core_axis_name="c", subcore_axis_name="s",
        num_cores=NC, num_subcores=NS)
    f = pl.kernel(
        _sc_body,
        out_type=jax.ShapeDtypeStruct((B, PAT), jnp.float32),
        mesh=mesh,
        compiler_params=pltpu.CompilerParams(needs_layout_passes=False),
        scratch_types=[
            pltpu.VMEM_SHARED((MEM,), jnp.int32),
            pltpu.VMEM((QW * K,), jnp.int32),
            pltpu.VMEM((4096,), jnp.int32),
            pltpu.VMEM((SPT,), jnp.int32),
            pltpu.VMEM((2 * L,), jnp.int32),
            pltpu.VMEM((2 * L,), jnp.float32),
        ] + [pltpu.VMEM((K,), jnp.int32)] * 3
          + [pltpu.VMEM((K + L, PAT), jnp.float32)] * 3
          + [pltpu.VMEM((K,), jnp.int32)] * 3
          + [pltpu.VMEM((L,), jnp.int32)] * 3
          + [pltpu.VMEM((K,), jnp.int32)] * 3
          + [pltpu.VMEM((PAT,), jnp.float32)] * 3
          + [pltpu.VMEM((PAT,), jnp.float32)] * 3
          + [pltpu.SemaphoreType.DMA] * 15,
    )
    return f(memory_patterns, patterns, write_idx, read_idx_flat)


def kernel(embeddings, memory_patterns, W_in, b_in, W_pat, b_pat,
           ln_gamma, ln_beta, write_idx, read_idx):
    patterns = _encode(
        embeddings, W_in, b_in.reshape(1, HID), W_pat, b_pat.reshape(1, PAT),
        ln_gamma.reshape(1, PAT), ln_beta.reshape(1, PAT))
    return _retrieve(
        memory_patterns, patterns,
        write_idx.astype(jnp.int32), read_idx.reshape(B * K).astype(jnp.int32))
